# baseline (speedup 1.0000x reference)
import functools

import jax
import jax.numpy as jnp
from jax import lax
from jax.experimental import pallas as pl
from jax.experimental.pallas import tpu as pltpu
from jax.experimental.pallas import tpu_sc as plsc

VOCAB = 1_000_000
DIM = 32
BATCH = 16384
SEQ = 50
PAD_IDX = 0
NUM_ROWS = BATCH * SEQ
CHUNK = 128
N_CHUNKS = NUM_ROWS // CHUNK
NC = 2
NS = 16
NW = NC * NS
CPW = N_CHUNKS // NW
NB = 8
LANES = 16
BBLK = BATCH // CHUNK
ETILES = DIM // 8


def _transpose_chunk(idx_row, rows_ref, t_ref):
    lane = jnp.arange(LANES, dtype=jnp.int32)
    masks = [
        idx_row[pl.ds(gi * LANES, LANES)] != PAD_IDX
        for gi in range(CHUNK // LANES)
    ]
    zero16 = jnp.zeros((LANES,), jnp.float32)
    for c in range(DIM):
        cvec = jnp.zeros((LANES,), jnp.int32) + c
        for gi in range(CHUNK // LANES):
            vals = plsc.load_gather(rows_ref, [lane + gi * LANES, cvec])
            vals = jnp.where(masks[gi], vals, zero16)
            t_ref[c, pl.ds(gi * LANES, LANES)] = vals


def _process_table(idx_hbm, w_hbm, out_hbm, idx_v, rows_v, t_v,
                   gsems, ssems, wid):
    base = wid * CPW
    pltpu.sync_copy(idx_hbm.at[pl.ds(base, CPW)], idx_v)

    def super_chunk(o, carry):
        for b in range(NB):
            lc = o * NB + b
            pltpu.async_copy(w_hbm.at[idx_v.at[lc]], rows_v.at[b], gsems[b])
        for b in range(NB):
            lc = o * NB + b
            pltpu.make_async_copy(
                w_hbm.at[idx_v.at[lc]], rows_v.at[b], gsems[b]
            ).wait()

        def transpose_buf(b, carry):
            lc = o * NB + b
            _transpose_chunk(idx_v.at[lc], rows_v.at[b], t_v.at[b])
            return carry

        lax.fori_loop(0, NB, transpose_buf, 0)
        for b in range(NB):
            lc = o * NB + b
            g = base + lc
            s = lax.shift_right_logical(g, 7)
            bb = lax.bitwise_and(g, CHUNK - 1)
            for et in range(ETILES):
                pltpu.async_copy(
                    t_v.at[b, pl.ds(et * 8, 8)],
                    out_hbm.at[s, et, bb],
                    ssems[b],
                )
        for b in range(NB):
            lc = o * NB + b
            g = base + lc
            s = lax.shift_right_logical(g, 7)
            bb = lax.bitwise_and(g, CHUNK - 1)
            for et in range(ETILES):
                pltpu.make_async_copy(
                    t_v.at[b, pl.ds(et * 8, 8)],
                    out_hbm.at[s, et, bb],
                    ssems[b],
                ).wait()
        return carry

    lax.fori_loop(0, CPW // NB, super_chunk, 0)


@functools.partial(
    pl.kernel,
    out_type=(
        jax.ShapeDtypeStruct((SEQ, ETILES, BBLK, 8, CHUNK), jnp.float32),
        jax.ShapeDtypeStruct((SEQ, ETILES, BBLK, 8, CHUNK), jnp.float32),
    ),
    mesh=plsc.VectorSubcoreMesh(core_axis_name="c", subcore_axis_name="s"),
    compiler_params=pltpu.CompilerParams(
        needs_layout_passes=False, use_tc_tiling_on_sc=False
    ),
    scratch_types=(
        [pltpu.VMEM((CPW, CHUNK), jnp.int32),
         pltpu.VMEM((NB, CHUNK, DIM), jnp.float32),
         pltpu.VMEM((NB, DIM, CHUNK), jnp.float32)]
        + [pltpu.SemaphoreType.DMA] * (2 * NB)
    ),
)
def _embed_sc(si_hbm, ti_hbm, sw_hbm, tw_hbm, so_hbm, to_hbm,
              idx_v, rows_v, t_v, *sems):
    wid = lax.axis_index("s") * NC + lax.axis_index("c")
    gsems, ssems = sems[:NB], sems[NB:]
    _process_table(si_hbm, sw_hbm, so_hbm, idx_v, rows_v, t_v,
                   gsems, ssems, wid)
    _process_table(ti_hbm, tw_hbm, to_hbm, idx_v, rows_v, t_v,
                   gsems, ssems, wid)


def kernel(src_indices, tgt_indices, src_weight, tgt_weight):
    si = jnp.transpose(src_indices).astype(jnp.int32).reshape(N_CHUNKS, CHUNK)
    ti = jnp.transpose(tgt_indices).astype(jnp.int32).reshape(N_CHUNKS, CHUNK)
    so5, to5 = _embed_sc(si, ti, src_weight, tgt_weight)
    so = so5.transpose(2, 4, 0, 1, 3).reshape(BATCH, SEQ, DIM)
    to = to5.transpose(2, 4, 0, 1, 3).reshape(BATCH, SEQ, DIM)
    return (so, to)


# speedup vs baseline: 1.6042x; 1.6042x over previous
import functools

import jax
import jax.numpy as jnp
from jax import lax
from jax.experimental import pallas as pl
from jax.experimental.pallas import tpu as pltpu
from jax.experimental.pallas import tpu_sc as plsc

VOCAB = 1_000_000
DIM = 32
BATCH = 16384
SEQ = 50
PAD_IDX = 0
NUM_ROWS = BATCH * SEQ
CHUNK = 128
N_CHUNKS = NUM_ROWS // CHUNK
NC = 2
NS = 16
NW = NC * NS
CPW = N_CHUNKS // NW
NB = 5
LANES = 16
BBLK = BATCH // CHUNK
ETILES = DIM // 8


def _transpose_chunk(idx_row, rows_ref, t_ref, lane, diags):
    zero16 = jnp.zeros((LANES,), jnp.float32)

    def k_block(kb, carry):
        kvec = lane + kb * LANES
        mk = plsc.load_gather(idx_row, [kvec]) != PAD_IDX
        for cb in range(DIM // LANES):
            for d in range(LANES):
                cvec = diags[d] + cb * LANES
                vals = plsc.load_gather(rows_ref, [kvec, cvec])
                vals = jnp.where(mk, vals, zero16)
                plsc.store_scatter(t_ref, [cvec, kvec], vals)
        return carry

    lax.fori_loop(0, CHUNK // LANES, k_block, 0)


def _process_table(idx_hbm, w_hbm, out_hbm, idx_v, rows_v, t_v,
                   gsems, ssems, wid):
    base = wid * CPW
    pltpu.sync_copy(idx_hbm.at[pl.ds(base, CPW)], idx_v)
    lane = jnp.arange(LANES, dtype=jnp.int32)
    diags = [
        jnp.bitwise_and(lane + d, LANES - 1) for d in range(LANES)
    ]

    def super_chunk(o, carry):
        for b in range(NB):
            lc = o * NB + b
            pltpu.async_copy(w_hbm.at[idx_v.at[lc]], rows_v.at[b], gsems[b])
        for b in range(NB):
            lc = o * NB + b
            pltpu.make_async_copy(
                w_hbm.at[idx_v.at[lc]], rows_v.at[b], gsems[b]
            ).wait()
            _transpose_chunk(idx_v.at[lc], rows_v.at[b], t_v.at[b],
                             lane, diags)
        for b in range(NB):
            lc = o * NB + b
            g = base + lc
            s = lax.shift_right_logical(g, 7)
            bb = lax.bitwise_and(g, CHUNK - 1)
            for et in range(ETILES):
                pltpu.async_copy(
                    t_v.at[b, pl.ds(et * 8, 8)],
                    out_hbm.at[s, et, bb],
                    ssems[b],
                )
        for b in range(NB):
            lc = o * NB + b
            g = base + lc
            s = lax.shift_right_logical(g, 7)
            bb = lax.bitwise_and(g, CHUNK - 1)
            for et in range(ETILES):
                pltpu.make_async_copy(
                    t_v.at[b, pl.ds(et * 8, 8)],
                    out_hbm.at[s, et, bb],
                    ssems[b],
                ).wait()
        return carry

    lax.fori_loop(0, CPW // NB, super_chunk, 0)


@functools.partial(
    pl.kernel,
    out_type=(
        jax.ShapeDtypeStruct((SEQ, ETILES, BBLK, 8, CHUNK), jnp.float32),
        jax.ShapeDtypeStruct((SEQ, ETILES, BBLK, 8, CHUNK), jnp.float32),
    ),
    mesh=plsc.VectorSubcoreMesh(core_axis_name="c", subcore_axis_name="s"),
    compiler_params=pltpu.CompilerParams(
        needs_layout_passes=False, use_tc_tiling_on_sc=False
    ),
    scratch_types=(
        [pltpu.VMEM((CPW, CHUNK), jnp.int32),
         pltpu.VMEM((NB, CHUNK, DIM), jnp.float32),
         pltpu.VMEM((NB, DIM, CHUNK), jnp.float32)]
        + [pltpu.SemaphoreType.DMA] * (2 * NB)
    ),
)
def _embed_sc(si_hbm, ti_hbm, sw_hbm, tw_hbm, so_hbm, to_hbm,
              idx_v, rows_v, t_v, *sems):
    wid = lax.axis_index("s") * NC + lax.axis_index("c")
    gsems, ssems = sems[:NB], sems[NB:]
    _process_table(si_hbm, sw_hbm, so_hbm, idx_v, rows_v, t_v,
                   gsems, ssems, wid)
    _process_table(ti_hbm, tw_hbm, to_hbm, idx_v, rows_v, t_v,
                   gsems, ssems, wid)


def kernel(src_indices, tgt_indices, src_weight, tgt_weight):
    si = jnp.transpose(src_indices).astype(jnp.int32).reshape(N_CHUNKS, CHUNK)
    ti = jnp.transpose(tgt_indices).astype(jnp.int32).reshape(N_CHUNKS, CHUNK)
    so5, to5 = _embed_sc(si, ti, src_weight, tgt_weight)
    so = so5.transpose(2, 4, 0, 1, 3).reshape(BATCH, SEQ, DIM)
    to = to5.transpose(2, 4, 0, 1, 3).reshape(BATCH, SEQ, DIM)
    return (so, to)


# speedup vs baseline: 1.6621x; 1.0361x over previous
import functools

import jax
import jax.numpy as jnp
from jax import lax
from jax.experimental import pallas as pl
from jax.experimental.pallas import tpu as pltpu
from jax.experimental.pallas import tpu_sc as plsc

VOCAB = 1_000_000
DIM = 32
BATCH = 16384
SEQ = 50
PAD_IDX = 0
NUM_ROWS = BATCH * SEQ
CHUNK = 128
N_CHUNKS = NUM_ROWS // CHUNK
NC = 2
NS = 16
NW = NC * NS
CPW = N_CHUNKS // NW
NB = 5
LANES = 16
BBLK = BATCH // CHUNK
ETILES = DIM // 8


def _transpose_chunk(idx_row, rows_ref, t_ref, lane, diags):
    zero16 = jnp.zeros((LANES,), jnp.float32)

    @plsc.parallel_loop(0, CHUNK // LANES)
    def k_block(kb):
        kvec = lane + kb * LANES
        mk = plsc.load_gather(idx_row, [kvec]) != PAD_IDX
        for cb in range(DIM // LANES):
            for d in range(LANES):
                cvec = diags[d] + cb * LANES
                vals = plsc.load_gather(rows_ref, [kvec, cvec])
                vals = jnp.where(mk, vals, zero16)
                plsc.store_scatter(t_ref, [cvec, kvec], vals)


def _process_table(idx_hbm, w_hbm, out_hbm, idx_v, rows_v, t_v,
                   gsems, ssems, wid):
    base = wid * CPW
    pltpu.sync_copy(idx_hbm.at[pl.ds(base, CPW)], idx_v)
    lane = jnp.arange(LANES, dtype=jnp.int32)
    diags = [
        jnp.bitwise_and(lane + d, LANES - 1) for d in range(LANES)
    ]

    def super_chunk(o, carry):
        for b in range(NB):
            lc = o * NB + b
            pltpu.async_copy(w_hbm.at[idx_v.at[lc]], rows_v.at[b], gsems[b])
        for b in range(NB):
            lc = o * NB + b
            pltpu.make_async_copy(
                w_hbm.at[idx_v.at[lc]], rows_v.at[b], gsems[b]
            ).wait()
            _transpose_chunk(idx_v.at[lc], rows_v.at[b], t_v.at[b],
                             lane, diags)
        for b in range(NB):
            lc = o * NB + b
            g = base + lc
            s = lax.shift_right_logical(g, 7)
            bb = lax.bitwise_and(g, CHUNK - 1)
            for et in range(ETILES):
                pltpu.async_copy(
                    t_v.at[b, pl.ds(et * 8, 8)],
                    out_hbm.at[s, et, bb],
                    ssems[b],
                )
        for b in range(NB):
            lc = o * NB + b
            g = base + lc
            s = lax.shift_right_logical(g, 7)
            bb = lax.bitwise_and(g, CHUNK - 1)
            for et in range(ETILES):
                pltpu.make_async_copy(
                    t_v.at[b, pl.ds(et * 8, 8)],
                    out_hbm.at[s, et, bb],
                    ssems[b],
                ).wait()
        return carry

    lax.fori_loop(0, CPW // NB, super_chunk, 0)


@functools.partial(
    pl.kernel,
    out_type=(
        jax.ShapeDtypeStruct((SEQ, ETILES, BBLK, 8, CHUNK), jnp.float32),
        jax.ShapeDtypeStruct((SEQ, ETILES, BBLK, 8, CHUNK), jnp.float32),
    ),
    mesh=plsc.VectorSubcoreMesh(core_axis_name="c", subcore_axis_name="s"),
    compiler_params=pltpu.CompilerParams(
        needs_layout_passes=False, use_tc_tiling_on_sc=False
    ),
    scratch_types=(
        [pltpu.VMEM((CPW, CHUNK), jnp.int32),
         pltpu.VMEM((NB, CHUNK, DIM), jnp.float32),
         pltpu.VMEM((NB, DIM, CHUNK), jnp.float32)]
        + [pltpu.SemaphoreType.DMA] * (2 * NB)
    ),
)
def _embed_sc(si_hbm, ti_hbm, sw_hbm, tw_hbm, so_hbm, to_hbm,
              idx_v, rows_v, t_v, *sems):
    wid = lax.axis_index("s") * NC + lax.axis_index("c")
    gsems, ssems = sems[:NB], sems[NB:]
    _process_table(si_hbm, sw_hbm, so_hbm, idx_v, rows_v, t_v,
                   gsems, ssems, wid)
    _process_table(ti_hbm, tw_hbm, to_hbm, idx_v, rows_v, t_v,
                   gsems, ssems, wid)


def kernel(src_indices, tgt_indices, src_weight, tgt_weight):
    si = jnp.transpose(src_indices).astype(jnp.int32).reshape(N_CHUNKS, CHUNK)
    ti = jnp.transpose(tgt_indices).astype(jnp.int32).reshape(N_CHUNKS, CHUNK)
    so5, to5 = _embed_sc(si, ti, src_weight, tgt_weight)
    so = so5.transpose(2, 4, 0, 1, 3).reshape(BATCH, SEQ, DIM)
    to = to5.transpose(2, 4, 0, 1, 3).reshape(BATCH, SEQ, DIM)
    return (so, to)


# speedup vs baseline: 1.7842x; 1.0734x over previous
import functools

import jax
import jax.numpy as jnp
from jax import lax
from jax.experimental import pallas as pl
from jax.experimental.pallas import tpu as pltpu
from jax.experimental.pallas import tpu_sc as plsc

VOCAB = 1_000_000
DIM = 32
BATCH = 16384
SEQ = 50
PAD_IDX = 0
NUM_ROWS = BATCH * SEQ
CHUNK = 128
N_CHUNKS = NUM_ROWS // CHUNK
NC = 2
NS = 16
NW = NC * NS
CPW = N_CHUNKS // NW
NB = 5
LANES = 16
BBLK = BATCH // CHUNK
ETILES = DIM // 8


def _transpose_chunk(idx_row, rows_ref, t_ref, lane, diags):
    zero16 = jnp.zeros((LANES,), jnp.float32)

    @plsc.parallel_loop(0, CHUNK // LANES)
    def k_block(kb):
        kvec = lane + kb * LANES
        mk = plsc.load_gather(idx_row, [kvec]) != PAD_IDX
        for cb in range(DIM // LANES):
            for d in range(LANES):
                cvec = diags[d] + cb * LANES
                vals = plsc.load_gather(rows_ref, [kvec, cvec])
                vals = jnp.where(mk, vals, zero16)
                plsc.store_scatter(t_ref, [cvec, kvec], vals)


def _process_table(idx_hbm, w_hbm, out_hbm, idx_v, rows_v, t_v,
                   gsems, ssems, wid):
    base = wid * CPW
    pltpu.sync_copy(idx_hbm.at[pl.ds(base, CPW)], idx_v)
    lane = jnp.arange(LANES, dtype=jnp.int32)
    diags = [
        jnp.bitwise_and(lane + d, LANES - 1) for d in range(LANES)
    ]

    def drain_scatters(o, b):
        lc = o * NB + b
        g = base + lc
        s = lax.shift_right_logical(g, 7)
        bb = lax.bitwise_and(g, CHUNK - 1)
        for et in range(ETILES):
            pltpu.make_async_copy(
                t_v.at[b, pl.ds(et * 8, 8)],
                out_hbm.at[s, et, bb],
                ssems[b],
            ).wait()

    def super_chunk(o, carry):
        for b in range(NB):
            lc = o * NB + b
            pltpu.async_copy(w_hbm.at[idx_v.at[lc]], rows_v.at[b], gsems[b])

        @pl.when(o > 0)
        def _():
            for b in range(NB):
                drain_scatters(o - 1, b)

        for b in range(NB):
            lc = o * NB + b
            pltpu.make_async_copy(
                w_hbm.at[idx_v.at[lc]], rows_v.at[b], gsems[b]
            ).wait()
            _transpose_chunk(idx_v.at[lc], rows_v.at[b], t_v.at[b],
                             lane, diags)
            g = base + lc
            s = lax.shift_right_logical(g, 7)
            bb = lax.bitwise_and(g, CHUNK - 1)
            for et in range(ETILES):
                pltpu.async_copy(
                    t_v.at[b, pl.ds(et * 8, 8)],
                    out_hbm.at[s, et, bb],
                    ssems[b],
                )
        return carry

    n_super = CPW // NB
    lax.fori_loop(0, n_super, super_chunk, 0)
    for b in range(NB):
        drain_scatters(n_super - 1, b)


@functools.partial(
    pl.kernel,
    out_type=(
        jax.ShapeDtypeStruct((SEQ, ETILES, BBLK, 8, CHUNK), jnp.float32),
        jax.ShapeDtypeStruct((SEQ, ETILES, BBLK, 8, CHUNK), jnp.float32),
    ),
    mesh=plsc.VectorSubcoreMesh(core_axis_name="c", subcore_axis_name="s"),
    compiler_params=pltpu.CompilerParams(
        needs_layout_passes=False, use_tc_tiling_on_sc=False
    ),
    scratch_types=(
        [pltpu.VMEM((CPW, CHUNK), jnp.int32),
         pltpu.VMEM((NB, CHUNK, DIM), jnp.float32),
         pltpu.VMEM((NB, DIM, CHUNK), jnp.float32)]
        + [pltpu.SemaphoreType.DMA] * (2 * NB)
    ),
)
def _embed_sc(si_hbm, ti_hbm, sw_hbm, tw_hbm, so_hbm, to_hbm,
              idx_v, rows_v, t_v, *sems):
    wid = lax.axis_index("s") * NC + lax.axis_index("c")
    gsems, ssems = sems[:NB], sems[NB:]
    _process_table(si_hbm, sw_hbm, so_hbm, idx_v, rows_v, t_v,
                   gsems, ssems, wid)
    _process_table(ti_hbm, tw_hbm, to_hbm, idx_v, rows_v, t_v,
                   gsems, ssems, wid)


def kernel(src_indices, tgt_indices, src_weight, tgt_weight):
    si = jnp.transpose(src_indices).astype(jnp.int32).reshape(N_CHUNKS, CHUNK)
    ti = jnp.transpose(tgt_indices).astype(jnp.int32).reshape(N_CHUNKS, CHUNK)
    so5, to5 = _embed_sc(si, ti, src_weight, tgt_weight)
    so = so5.transpose(2, 4, 0, 1, 3).reshape(BATCH, SEQ, DIM)
    to = to5.transpose(2, 4, 0, 1, 3).reshape(BATCH, SEQ, DIM)
    return (so, to)


# speedup vs baseline: 2.0429x; 1.1450x over previous
import functools

import jax
import jax.numpy as jnp
from jax import lax
from jax.experimental import pallas as pl
from jax.experimental.pallas import tpu as pltpu
from jax.experimental.pallas import tpu_sc as plsc

VOCAB = 1_000_000
DIM = 32
BATCH = 16384
SEQ = 50
PAD_IDX = 0
NUM_ROWS = BATCH * SEQ
CHUNK = 128
N_CHUNKS = NUM_ROWS // CHUNK
NC = 2
NS = 16
NW = NC * NS
CPW = N_CHUNKS // NW
NB = 8
LANES = 16
BBLK = BATCH // CHUNK
ETILES = DIM // 8


def _transpose_chunk(idx_row, rows_ref, t_ref, lane, diags):
    zero16 = jnp.zeros((LANES,), jnp.float32)

    @plsc.parallel_loop(0, CHUNK // LANES)
    def k_block(kb):
        kvec = lane + kb * LANES
        mk = plsc.load_gather(idx_row, [kvec]) != PAD_IDX
        for cb in range(DIM // LANES):
            for d in range(LANES):
                cvec = diags[d] + cb * LANES
                vals = plsc.load_gather(rows_ref, [kvec, cvec])
                vals = jnp.where(mk, vals, zero16)
                plsc.store_scatter(t_ref, [cvec, kvec], vals)


def _process_table(idx_hbm, w_hbm, out_hbm, idx_v, rows_v, t_v,
                   gsems, ssems, wid):
    base = wid * CPW
    pltpu.sync_copy(idx_hbm.at[pl.ds(base, CPW)], idx_v)
    lane = jnp.arange(LANES, dtype=jnp.int32)
    diags = [
        jnp.bitwise_and(lane + d, LANES - 1) for d in range(LANES)
    ]

    def drain_scatters(o, b):
        lc = o * NB + b
        g = base + lc
        s = lax.shift_right_logical(g, 7)
        bb = lax.bitwise_and(g, CHUNK - 1)
        for et in range(ETILES):
            pltpu.make_async_copy(
                t_v.at[b, pl.ds(et * 8, 8)],
                out_hbm.at[s, et, bb],
                ssems[b],
            ).wait()

    def super_chunk(o, carry):
        for b in range(NB):
            lc = o * NB + b
            pltpu.async_copy(w_hbm.at[idx_v.at[lc]], rows_v.at[b], gsems[b])

        @pl.when(o > 0)
        def _():
            for b in range(NB):
                drain_scatters(o - 1, b)

        for b in range(NB):
            lc = o * NB + b
            pltpu.make_async_copy(
                w_hbm.at[idx_v.at[lc]], rows_v.at[b], gsems[b]
            ).wait()
            _transpose_chunk(idx_v.at[lc], rows_v.at[b], t_v.at[b],
                             lane, diags)
            g = base + lc
            s = lax.shift_right_logical(g, 7)
            bb = lax.bitwise_and(g, CHUNK - 1)
            for et in range(ETILES):
                pltpu.async_copy(
                    t_v.at[b, pl.ds(et * 8, 8)],
                    out_hbm.at[s, et, bb],
                    ssems[b],
                )
        return carry

    n_super = CPW // NB
    lax.fori_loop(0, n_super, super_chunk, 0)
    for b in range(NB):
        drain_scatters(n_super - 1, b)


@functools.partial(
    pl.kernel,
    out_type=jax.ShapeDtypeStruct((SEQ, ETILES, BBLK, 8, CHUNK), jnp.float32),
    mesh=plsc.VectorSubcoreMesh(core_axis_name="c", subcore_axis_name="s"),
    compiler_params=pltpu.CompilerParams(
        needs_layout_passes=False, use_tc_tiling_on_sc=False
    ),
    scratch_types=(
        [pltpu.VMEM((CPW, CHUNK), jnp.int32),
         pltpu.VMEM((NB, CHUNK, DIM), jnp.float32),
         pltpu.VMEM((NB, DIM, CHUNK), jnp.float32)]
        + [pltpu.SemaphoreType.DMA] * (2 * NB)
    ),
)
def _embed_sc(i_hbm, w_hbm, o_hbm, idx_v, rows_v, t_v, *sems):
    wid = lax.axis_index("s") * NC + lax.axis_index("c")
    gsems, ssems = sems[:NB], sems[NB:]
    _process_table(i_hbm, w_hbm, o_hbm, idx_v, rows_v, t_v,
                   gsems, ssems, wid)


def kernel(src_indices, tgt_indices, src_weight, tgt_weight):
    si = jnp.transpose(src_indices).astype(jnp.int32).reshape(N_CHUNKS, CHUNK)
    ti = jnp.transpose(tgt_indices).astype(jnp.int32).reshape(N_CHUNKS, CHUNK)
    so5 = _embed_sc(si, src_weight)
    to5 = _embed_sc(ti, tgt_weight)
    so = so5.transpose(2, 4, 0, 1, 3).reshape(BATCH, SEQ, DIM)
    to = to5.transpose(2, 4, 0, 1, 3).reshape(BATCH, SEQ, DIM)
    return (so, to)
